# baseline (speedup 1.0000x reference)
import functools

import jax
import jax.numpy as jnp
from jax.experimental import pallas as pl
from jax.experimental.pallas import tpu as pltpu

_C = 0.7978845608028654
_A = 0.044715


def _gelu(h):
    u = h * h
    p = u * (_A * _C) + _C
    w = jnp.tanh(h * p)
    q = h * 0.5
    return q * w + q


def _mlp_body(col_ref, x_ref, w1_ref, b1_ref, w2_ref, b2_ref, o_ref,
              x16_ref, *, bf, sub):
    j = pl.program_id(1)

    @pl.when(j == 0)
    def _cast_x():
        x16_ref[...] = x_ref[...].astype(jnp.bfloat16)

    x = x16_ref[...]
    nsub = bf // sub
    acc = None
    for k in range(nsub):
        sl = slice(k * sub, (k + 1) * sub)
        hk = jnp.dot(x, w1_ref[0, :, sl].astype(jnp.bfloat16),
                     preferred_element_type=jnp.float32)
        gk = _gelu(hk + b1_ref[0, 0, 0, sl]).astype(jnp.bfloat16)
        ak = jnp.dot(gk, w2_ref[0, sl, :].astype(jnp.bfloat16),
                     preferred_element_type=jnp.float32)
        acc = ak if acc is None else acc + ak

    @pl.when(j == 0)
    def _init():
        o_ref[...] = acc + b2_ref[0, 0]

    @pl.when(j != 0)
    def _accum():
        o_ref[...] += acc


@functools.partial(jax.jit, static_argnames=("bt", "bf", "sub"))
def _moe_mlp(hidden_states, W1, b1, W2, b2, col, bt=1024, bf=1024, sub=256):
    T, D = hidden_states.shape
    E, _, F = W1.shape
    col_arr = jnp.atleast_1d(jnp.asarray(col, jnp.int32))
    b1r = b1.reshape(E, F // bf, 1, bf)
    b2r = b2.reshape(E, 1, 1, D)

    grid = (T // bt, F // bf)
    grid_spec = pltpu.PrefetchScalarGridSpec(
        num_scalar_prefetch=1,
        grid=grid,
        in_specs=[
            pl.BlockSpec((bt, D), lambda i, j, c: (i, 0)),
            pl.BlockSpec((1, D, bf), lambda i, j, c: (c[0], 0, j)),
            pl.BlockSpec((1, 1, 1, bf), lambda i, j, c: (c[0], j, 0, 0)),
            pl.BlockSpec((1, bf, D), lambda i, j, c: (c[0], j, 0)),
            pl.BlockSpec((1, 1, 1, D), lambda i, j, c: (c[0], 0, 0, 0)),
        ],
        out_specs=pl.BlockSpec((bt, D), lambda i, j, c: (i, 0)),
        scratch_shapes=[pltpu.VMEM((bt, D), jnp.bfloat16)],
    )
    body = functools.partial(_mlp_body, bf=bf, sub=sub)
    return pl.pallas_call(
        body,
        grid_spec=grid_spec,
        out_shape=jax.ShapeDtypeStruct((T, D), jnp.float32),
        compiler_params=pltpu.CompilerParams(
            dimension_semantics=("parallel", "arbitrary"),
        ),
    )(col_arr, hidden_states, W1, b1r, W2, b2r)


def kernel(hidden_states, W1, b1, W2, b2, col):
    return _moe_mlp(hidden_states, W1, b1, W2, b2, col)


# speedup vs baseline: 1.1987x; 1.1987x over previous
import functools

import jax
import jax.numpy as jnp
from jax.experimental import pallas as pl
from jax.experimental.pallas import tpu as pltpu


def _mlp_body(col_ref, x_ref, w1_ref, b1_ref, w2_ref, b2_ref, o_ref,
              x16_ref, w1b_ref, w2b_ref):
    i = pl.program_id(0)
    j = pl.program_id(1)

    @pl.when(i == 0)
    def _cache_w():
        w1b_ref[j] = w1_ref[0].astype(jnp.bfloat16)
        w2b_ref[j] = w2_ref[0].astype(jnp.bfloat16)

    @pl.when(j == 0)
    def _cache_x():
        x16_ref[...] = x_ref[...].astype(jnp.bfloat16)

    h = jnp.dot(x16_ref[...], w1b_ref[j],
                preferred_element_type=jnp.float32)
    h = jax.nn.gelu(h + b1_ref[0, 0]).astype(jnp.bfloat16)
    acc = jnp.dot(h, w2b_ref[j], preferred_element_type=jnp.float32)

    @pl.when(j == 0)
    def _init():
        o_ref[...] = acc + b2_ref[0, 0]

    @pl.when(j != 0)
    def _accum():
        o_ref[...] += acc


@functools.partial(jax.jit, static_argnames=("bt", "bf"))
def _moe_mlp(hidden_states, W1, b1, W2, b2, col, bt=1024, bf=1024):
    T, D = hidden_states.shape
    E, _, F = W1.shape
    nff = F // bf
    col_arr = jnp.atleast_1d(jnp.asarray(col, jnp.int32))
    b1r = b1.reshape(E, nff, 1, bf)
    b2r = b2.reshape(E, 1, 1, D)

    grid = (T // bt, nff)
    grid_spec = pltpu.PrefetchScalarGridSpec(
        num_scalar_prefetch=1,
        grid=grid,
        in_specs=[
            pl.BlockSpec((bt, D), lambda i, j, c: (i, 0)),
            pl.BlockSpec((1, D, bf), lambda i, j, c: (c[0], 0, j)),
            pl.BlockSpec((1, 1, 1, bf), lambda i, j, c: (c[0], j, 0, 0)),
            pl.BlockSpec((1, bf, D), lambda i, j, c: (c[0], j, 0)),
            pl.BlockSpec((1, 1, 1, D), lambda i, j, c: (c[0], 0, 0, 0)),
        ],
        out_specs=pl.BlockSpec((bt, D), lambda i, j, c: (i, 0)),
        scratch_shapes=[
            pltpu.VMEM((bt, D), jnp.bfloat16),
            pltpu.VMEM((nff, D, bf), jnp.bfloat16),
            pltpu.VMEM((nff, bf, D), jnp.bfloat16),
        ],
    )
    return pl.pallas_call(
        _mlp_body,
        grid_spec=grid_spec,
        out_shape=jax.ShapeDtypeStruct((T, D), jnp.float32),
        compiler_params=pltpu.CompilerParams(
            dimension_semantics=("parallel", "arbitrary"),
        ),
    )(col_arr, hidden_states, W1, b1r, W2, b2r)


def kernel(hidden_states, W1, b1, W2, b2, col):
    return _moe_mlp(hidden_states, W1, b1, W2, b2, col)
